# baseline (speedup 1.0000x reference)
import functools

import jax
import jax.numpy as jnp
from jax import lax
from jax.experimental import pallas as pl
from jax.experimental.pallas import tpu as pltpu
from jax.experimental.pallas import tpu_sc as plsc

NC, NS, LANES = 2, 16, 16
NW = NC * NS


def _sc_mesh():
    return plsc.VectorSubcoreMesh(
        core_axis_name="c", subcore_axis_name="s", num_cores=NC, num_subcores=NS
    )


def _wid():
    return lax.axis_index("s") * NC + lax.axis_index("c")


_BR = 256


def _scorank_body(h_ref, w_ref, b_ref, rank_ref, hs_ref):
    hv = h_ref[...]
    w = w_ref[...]
    wt_full = lax.dot_general(hv, w, (((1,), (1,)), ((), ())))
    wt = wt_full[:, 0:1]
    s = jax.nn.sigmoid(wt + b_ref[0])
    hs_ref[...] = hv * s
    n = s.shape[0]
    sr = jnp.transpose(s)
    for bi in range(n // _BR):
        lo = bi * _BR
        sc = s[lo : lo + _BR, :]
        cnt = jnp.zeros((_BR, 1), jnp.float32)
        if lo > 0:
            pre = sr[:, :lo]
            cnt += jnp.sum((pre >= sc).astype(jnp.float32), axis=1, keepdims=True)
        if lo + _BR < n:
            post = sr[:, lo + _BR :]
            cnt += jnp.sum((post > sc).astype(jnp.float32), axis=1, keepdims=True)
        diag = sr[:, lo : lo + _BR]
        jrow = lax.broadcasted_iota(jnp.int32, (_BR, _BR), 1)
        irow = lax.broadcasted_iota(jnp.int32, (_BR, _BR), 0)
        dcnt = jnp.where(jrow < irow, (diag >= sc).astype(jnp.float32), 0.0) + (
            jnp.where(jrow >= irow, (diag > sc).astype(jnp.float32), 0.0)
        )
        cnt += jnp.sum(dcnt, axis=1, keepdims=True)
        rank_ref[lo : lo + _BR, :] = cnt.astype(jnp.int32)


def _scorank_tc(h, W2, b1):
    N, D = h.shape
    return pl.pallas_call(
        _scorank_body,
        in_specs=[
            pl.BlockSpec(memory_space=pltpu.MemorySpace.VMEM),
            pl.BlockSpec(memory_space=pltpu.MemorySpace.VMEM),
            pl.BlockSpec(memory_space=pltpu.MemorySpace.SMEM),
        ],
        out_shape=[
            jax.ShapeDtypeStruct((N, 1), jnp.int32),
            jax.ShapeDtypeStruct((N, D), jnp.float32),
        ],
    )(h, W2, b1)


def _selgather_sc(rank1, g, hs, kk, lo0):
    N = g.shape[0]
    D = hs.shape[1]
    per = kk // NW
    ch = 8
    nch = per // ch

    @functools.partial(
        pl.kernel,
        out_type=[
            jax.ShapeDtypeStruct((kk,), jnp.int32),
            jax.ShapeDtypeStruct((kk, N), jnp.float32),
            jax.ShapeDtypeStruct((kk, D), jnp.float32),
        ],
        mesh=_sc_mesh(),
        compiler_params=pltpu.CompilerParams(needs_layout_passes=False),
        scratch_types=[
            pltpu.VMEM((N,), jnp.int32),
            pltpu.VMEM((per,), jnp.int32),
            pltpu.VMEM((2, ch, N), jnp.float32),
            pltpu.VMEM((per, D), jnp.float32),
            pltpu.SemaphoreType.DMA,
            pltpu.SemaphoreType.DMA,
            pltpu.SemaphoreType.DMA,
        ],
    )
    def gat(
        rank_hbm, g_hbm, hs_hbm, idx_hbm, a_hbm, nh_hbm,
        rank_v, idx_v, grow_v, hrow_v, s0, s1, hsem,
    ):
        base = _wid() * per
        rank_lo = lo0 + base
        pltpu.sync_copy(rank_hbm, rank_v)

        def body(c, carry):
            r = rank_v[pl.ds(c * LANES, LANES)]
            iv = lax.iota(jnp.int32, LANES) + c * LANES
            m = (r >= rank_lo) & (r < rank_lo + per)
            rr = jnp.where(m, r - rank_lo, 0)
            plsc.store_scatter(idx_v, [rr], iv, mask=m)
            return carry

        lax.fori_loop(0, N // LANES, body, 0)
        pltpu.sync_copy(idx_v, idx_hbm.at[pl.ds(base, per)])

        hcp = pltpu.async_copy(hs_hbm.at[idx_v], hrow_v, hsem)
        sems = (s0, s1)
        cps = [None] * nch
        cps[0] = pltpu.async_copy(g_hbm.at[idx_v.at[pl.ds(0, ch)]], grow_v.at[0], sems[0])
        for t in range(nch):
            if t + 1 < nch:
                cps[t + 1] = pltpu.async_copy(
                    g_hbm.at[idx_v.at[pl.ds((t + 1) * ch, ch)]],
                    grow_v.at[(t + 1) % 2],
                    sems[(t + 1) % 2],
                )
            cps[t].wait()
            pltpu.sync_copy(grow_v.at[t % 2], a_hbm.at[pl.ds(base + t * ch, ch)])
        hcp.wait()
        pltpu.sync_copy(hrow_v, nh_hbm.at[pl.ds(base, per)])

    return gat(rank1, g, hs)


_BJ = 256


def _mm_body(a_ref, g_ref, o_ref):
    acc = jnp.dot(a_ref[...], g_ref[...], preferred_element_type=jnp.float32)
    o_ref[...] = (jnp.transpose(acc) > 0.0).astype(jnp.float32)


def _matmul_tc(A, g):
    kk, N = A.shape
    return pl.pallas_call(
        _mm_body,
        grid=(N // _BJ,),
        in_specs=[
            pl.BlockSpec((kk, N), lambda j: (0, 0)),
            pl.BlockSpec((N, _BJ), lambda j: (0, j)),
        ],
        out_specs=pl.BlockSpec((_BJ, kk), lambda j: (j, 0)),
        out_shape=jax.ShapeDtypeStruct((N, kk), jnp.float32),
    )(A, g)


def _rowgather2_sc(MT1, MT2, idx1, idx2):
    half = idx1.shape[0]
    kk = 2 * half
    w = MT1.shape[1]
    per = kk // NW
    ch = 32
    nch = per // ch

    @functools.partial(
        pl.kernel,
        out_type=jax.ShapeDtypeStruct((kk, kk), MT1.dtype),
        mesh=_sc_mesh(),
        compiler_params=pltpu.CompilerParams(needs_layout_passes=False),
        scratch_types=[
            pltpu.VMEM((per,), jnp.int32),
            pltpu.VMEM((ch, w), MT1.dtype),
            pltpu.VMEM((ch, w), MT1.dtype),
            pltpu.SemaphoreType.DMA,
            pltpu.SemaphoreType.DMA,
        ],
    )
    def rg(mt1_hbm, mt2_hbm, idx1_hbm, idx2_hbm, out_hbm, idx_v, r1_v, r2_v, s1, s2):
        wid = _wid()
        base = wid * per
        nhalf = NW // 2

        @pl.when(wid < nhalf)
        def _():
            pltpu.sync_copy(idx1_hbm.at[pl.ds(base, per)], idx_v)

        @pl.when(wid >= nhalf)
        def _():
            pltpu.sync_copy(idx2_hbm.at[pl.ds((wid - nhalf) * per, per)], idx_v)

        for t in range(nch):
            sl = idx_v.at[pl.ds(t * ch, ch)]
            c1 = pltpu.async_copy(mt1_hbm.at[sl], r1_v, s1)
            c2 = pltpu.async_copy(mt2_hbm.at[sl], r2_v, s2)
            c1.wait()
            c2.wait()
            rows = pl.ds(base + t * ch, ch)
            pltpu.sync_copy(r1_v, out_hbm.at[rows, pl.ds(0, w)])
            pltpu.sync_copy(r2_v, out_hbm.at[rows, pl.ds(w, w)])

    return rg(MT1, MT2, idx1, idx2)


_BT = 512


def _transpose_body(i_ref, o_ref):
    o_ref[...] = jnp.transpose(i_ref[...]).astype(jnp.float32)


def _transpose_tc(X):
    kk = X.shape[0]
    nb = kk // _BT
    return pl.pallas_call(
        _transpose_body,
        grid=(nb, nb),
        in_specs=[pl.BlockSpec((_BT, _BT), lambda i, j: (j, i))],
        out_specs=pl.BlockSpec((_BT, _BT), lambda i, j: (i, j)),
        out_shape=jax.ShapeDtypeStruct((kk, kk), jnp.float32),
    )(X)


def kernel(g, h, ep, W, b):
    N, D = h.shape
    kk = max(2, N // 2)
    Wp = jnp.pad(W, ((0, 127), (0, 0)))
    rank, hs = _scorank_tc(h, Wp, b)
    rank1 = rank.reshape(N)
    half = kk // 2
    idx1, A1, nh1 = _selgather_sc(rank1, g, hs, half, 0)
    MT1 = _matmul_tc(A1, g)
    idx2, A2, nh2 = _selgather_sc(rank1, g, hs, half, half)
    MT2 = _matmul_tc(A2, g)
    un_gT = _rowgather2_sc(MT1, MT2, idx1, idx2)
    un_g = _transpose_tc(un_gT)
    idx = jnp.concatenate([idx1, idx2])
    new_h = jnp.concatenate([nh1, nh2])
    return un_g, new_h, idx


# speedup vs baseline: 1.0328x; 1.0328x over previous
import functools

import jax
import jax.numpy as jnp
from jax import lax
from jax.experimental import pallas as pl
from jax.experimental.pallas import tpu as pltpu
from jax.experimental.pallas import tpu_sc as plsc

NC, NS, LANES = 2, 16, 16
NW = NC * NS


def _sc_mesh():
    return plsc.VectorSubcoreMesh(
        core_axis_name="c", subcore_axis_name="s", num_cores=NC, num_subcores=NS
    )


def _wid():
    return lax.axis_index("s") * NC + lax.axis_index("c")


_BR = 256


def _scorank_body(h_ref, w_ref, b_ref, rank_ref, hs_ref):
    hv = h_ref[...]
    w = w_ref[...]
    wt_full = lax.dot_general(hv, w, (((1,), (1,)), ((), ())))
    wt = wt_full[:, 0:1]
    s = jax.nn.sigmoid(wt + b_ref[0])
    hs_ref[...] = hv * s
    n = s.shape[0]
    sr = jnp.transpose(s)
    for bi in range(n // _BR):
        lo = bi * _BR
        sc = s[lo : lo + _BR, :]
        cnt = jnp.zeros((_BR, 1), jnp.float32)
        if lo > 0:
            pre = sr[:, :lo]
            cnt += jnp.sum((pre >= sc).astype(jnp.float32), axis=1, keepdims=True)
        if lo + _BR < n:
            post = sr[:, lo + _BR :]
            cnt += jnp.sum((post > sc).astype(jnp.float32), axis=1, keepdims=True)
        diag = sr[:, lo : lo + _BR]
        jrow = lax.broadcasted_iota(jnp.int32, (_BR, _BR), 1)
        irow = lax.broadcasted_iota(jnp.int32, (_BR, _BR), 0)
        dcnt = jnp.where(jrow < irow, (diag >= sc).astype(jnp.float32), 0.0) + (
            jnp.where(jrow >= irow, (diag > sc).astype(jnp.float32), 0.0)
        )
        cnt += jnp.sum(dcnt, axis=1, keepdims=True)
        rank_ref[lo : lo + _BR, :] = cnt.astype(jnp.int32)


def _scorank_tc(h, W2, b1):
    N, D = h.shape
    return pl.pallas_call(
        _scorank_body,
        in_specs=[
            pl.BlockSpec(memory_space=pltpu.MemorySpace.VMEM),
            pl.BlockSpec(memory_space=pltpu.MemorySpace.VMEM),
            pl.BlockSpec(memory_space=pltpu.MemorySpace.SMEM),
        ],
        out_shape=[
            jax.ShapeDtypeStruct((N, 1), jnp.int32),
            jax.ShapeDtypeStruct((N, D), jnp.float32),
        ],
    )(h, W2, b1)


def _selgather_sc(rank1, g, hs, kk):
    N = g.shape[0]
    D = hs.shape[1]
    per = kk // NW
    ch = 8
    nch = per // ch

    @functools.partial(
        pl.kernel,
        out_type=[
            jax.ShapeDtypeStruct((kk,), jnp.int32),
            jax.ShapeDtypeStruct((kk, N), jnp.float32),
            jax.ShapeDtypeStruct((kk, D), jnp.float32),
        ],
        mesh=_sc_mesh(),
        compiler_params=pltpu.CompilerParams(needs_layout_passes=False),
        scratch_types=[
            pltpu.VMEM((N,), jnp.int32),
            pltpu.VMEM((per,), jnp.int32),
            pltpu.VMEM((2, ch, N), jnp.float32),
            pltpu.VMEM((per, D), jnp.float32),
            pltpu.SemaphoreType.DMA,
            pltpu.SemaphoreType.DMA,
            pltpu.SemaphoreType.DMA,
        ],
    )
    def gat(
        rank_hbm, g_hbm, hs_hbm, idx_hbm, a_hbm, nh_hbm,
        rank_v, idx_v, grow_v, hrow_v, s0, s1, hsem,
    ):
        base = _wid() * per
        pltpu.sync_copy(rank_hbm, rank_v)

        def body(c, carry):
            r = rank_v[pl.ds(c * LANES, LANES)]
            iv = lax.iota(jnp.int32, LANES) + c * LANES
            m = (r >= base) & (r < base + per)
            rr = jnp.where(m, r - base, 0)
            plsc.store_scatter(idx_v, [rr], iv, mask=m)
            return carry

        lax.fori_loop(0, N // LANES, body, 0)
        pltpu.sync_copy(idx_v, idx_hbm.at[pl.ds(base, per)])

        hcp = pltpu.async_copy(hs_hbm.at[idx_v], hrow_v, hsem)
        sems = (s0, s1)
        cps = [None] * nch
        cps[0] = pltpu.async_copy(g_hbm.at[idx_v.at[pl.ds(0, ch)]], grow_v.at[0], sems[0])
        for t in range(nch):
            if t + 1 < nch:
                cps[t + 1] = pltpu.async_copy(
                    g_hbm.at[idx_v.at[pl.ds((t + 1) * ch, ch)]],
                    grow_v.at[(t + 1) % 2],
                    sems[(t + 1) % 2],
                )
            cps[t].wait()
            pltpu.sync_copy(grow_v.at[t % 2], a_hbm.at[pl.ds(base + t * ch, ch)])
        hcp.wait()
        pltpu.sync_copy(hrow_v, nh_hbm.at[pl.ds(base, per)])

    return gat(rank1, g, hs)


_BJ = 256


def _mm_body(a_ref, g_ref, o_ref):
    acc = jnp.dot(a_ref[...], g_ref[...], preferred_element_type=jnp.float32)
    o_ref[...] = (jnp.transpose(acc) > 0.0).astype(jnp.float32)


def _matmul_tc(A, g):
    kk, N = A.shape
    return pl.pallas_call(
        _mm_body,
        grid=(N // _BJ,),
        in_specs=[
            pl.BlockSpec((kk, N), lambda j: (0, 0)),
            pl.BlockSpec((N, _BJ), lambda j: (0, j)),
        ],
        out_specs=pl.BlockSpec((_BJ, kk), lambda j: (j, 0)),
        out_shape=jax.ShapeDtypeStruct((N, kk), jnp.float32),
    )(A, g)


def _rowgather_sc(MT, idx):
    kk = idx.shape[0]
    w = MT.shape[1]
    per = kk // NW
    ch = LANES
    nch = per // ch

    @functools.partial(
        pl.kernel,
        out_type=jax.ShapeDtypeStruct((kk, w), MT.dtype),
        mesh=_sc_mesh(),
        compiler_params=pltpu.CompilerParams(needs_layout_passes=False),
        scratch_types=[
            pltpu.VMEM((per,), jnp.int32),
            pltpu.VMEM((2, ch, w), MT.dtype),
            pltpu.SemaphoreType.DMA,
            pltpu.SemaphoreType.DMA,
        ],
    )
    def rg(mt_hbm, idx_hbm, out_hbm, idx_v, rows_v, sem0, sem1):
        base = _wid() * per
        pltpu.sync_copy(idx_hbm.at[pl.ds(base, per)], idx_v)
        sems = (sem0, sem1)
        cps = [None] * nch
        cps[0] = pltpu.async_copy(
            mt_hbm.at[idx_v.at[pl.ds(0, ch)]], rows_v.at[0], sems[0]
        )
        for t in range(nch):
            if t + 1 < nch:
                cps[t + 1] = pltpu.async_copy(
                    mt_hbm.at[idx_v.at[pl.ds((t + 1) * ch, ch)]],
                    rows_v.at[(t + 1) % 2],
                    sems[(t + 1) % 2],
                )
            cps[t].wait()
            pltpu.sync_copy(rows_v.at[t % 2], out_hbm.at[pl.ds(base + t * ch, ch)])

    return rg(MT, idx)


_BT = 512


def _transpose_body(i_ref, o_ref):
    o_ref[...] = jnp.transpose(i_ref[...]).astype(jnp.float32)


def _transpose_tc(X):
    kk = X.shape[0]
    nb = kk // _BT
    return pl.pallas_call(
        _transpose_body,
        grid=(nb, nb),
        in_specs=[pl.BlockSpec((_BT, _BT), lambda i, j: (j, i))],
        out_specs=pl.BlockSpec((_BT, _BT), lambda i, j: (i, j)),
        out_shape=jax.ShapeDtypeStruct((kk, kk), jnp.float32),
    )(X)


def kernel(g, h, ep, W, b):
    N, D = h.shape
    kk = max(2, N // 2)
    Wp = jnp.pad(W, ((0, 127), (0, 0)))
    rank, hs = _scorank_tc(h, Wp, b)
    idx, A, new_h = _selgather_sc(rank.reshape(N), g, hs, kk)
    MT = _matmul_tc(A, g)
    un_gT = _rowgather_sc(MT, idx)
    un_g = _transpose_tc(un_gT)
    return un_g, new_h, idx
